# baseline (speedup 1.0000x reference)
import functools

import jax
import jax.numpy as jnp
from jax.experimental import pallas as pl

N = 131072
D = 32
K = 512
BN = 1024


def _nc_kernel(x_ref, ct_ref, csq_ref, out_ref):
    x = x_ref[...]
    ct = ct_ref[...]
    csq = csq_ref[...]
    s = jax.lax.dot_general(
        x, ct, (((1,), (0,)), ((), ())),
        preferred_element_type=jnp.float32,
    )
    scores = csq - 2.0 * s
    m = jnp.min(scores, axis=1, keepdims=True)
    ids = jax.lax.broadcasted_iota(jnp.int32, scores.shape, 1)
    idx = jnp.min(jnp.where(scores == m, ids, K), axis=1)
    out_ref[...] = idx.astype(jnp.int32)


@functools.partial(jax.jit, static_argnames=())
def kernel(X, centroids):
    ct = centroids.T
    csq = jnp.sum(centroids * centroids, axis=1)[None, :]
    out = pl.pallas_call(
        _nc_kernel,
        grid=(N // BN,),
        in_specs=[
            pl.BlockSpec((BN, D), lambda i: (i, 0)),
            pl.BlockSpec((D, K), lambda i: (0, 0)),
            pl.BlockSpec((1, K), lambda i: (0, 0)),
        ],
        out_specs=pl.BlockSpec((BN,), lambda i: (i,)),
        out_shape=jax.ShapeDtypeStruct((N,), jnp.int32),
    )(X, ct, csq)
    return out


# speedup vs baseline: 2.6042x; 2.6042x over previous
import jax
import jax.numpy as jnp
from jax.experimental import pallas as pl

N = 131072
D = 32
K = 512
DP = 40
BN = 1024


def _nc_kernel(a_ref, xt_ref, out_ref):
    a = a_ref[...]
    xt = xt_ref[...]
    s = jax.lax.dot_general(
        a, xt, (((1,), (0,)), ((), ())),
        preferred_element_type=jnp.float32,
    )
    m = jnp.min(s, axis=0, keepdims=True)
    ids = jax.lax.broadcasted_iota(jnp.int32, (K, 1), 0).astype(jnp.float32)
    idxf = jnp.min(jnp.where(s == m, ids, float(K)), axis=0)
    out_ref[...] = idxf.astype(jnp.int32)


def kernel(X, centroids):
    csq = jnp.sum(centroids * centroids, axis=1, keepdims=True)
    a = jnp.concatenate(
        [-2.0 * centroids, csq,
         jnp.zeros((K, DP - D - 1), jnp.float32)], axis=1)
    xt = jnp.concatenate(
        [X.T, jnp.ones((1, N), jnp.float32),
         jnp.zeros((DP - D - 1, N), jnp.float32)], axis=0)
    out = pl.pallas_call(
        _nc_kernel,
        grid=(N // BN,),
        in_specs=[
            pl.BlockSpec((K, DP), lambda i: (0, 0)),
            pl.BlockSpec((DP, BN), lambda i: (0, i)),
        ],
        out_specs=pl.BlockSpec((BN,), lambda i: (i,)),
        out_shape=jax.ShapeDtypeStruct((N,), jnp.int32),
    )(a, xt)
    return out


# speedup vs baseline: 2.6149x; 1.0041x over previous
import jax
import jax.numpy as jnp
from jax.experimental import pallas as pl

N = 131072
D = 32
K = 512
DP = 32
BN = 1024


def _nc_kernel(a_ref, xt_ref, csq_ref, xsq_ref, out_ref):
    a = a_ref[...]
    xt = xt_ref[...]
    s = jax.lax.dot_general(
        a, xt, (((1,), (0,)), ((), ())),
        preferred_element_type=jnp.float32,
    )
    t = xsq_ref[...] + csq_ref[...]
    d2 = t + s
    m = jnp.min(d2, axis=0, keepdims=True)
    ids = jax.lax.broadcasted_iota(jnp.int32, (K, 1), 0).astype(jnp.float32)
    idxf = jnp.min(jnp.where(d2 == m, ids, float(K)), axis=0)
    out_ref[...] = idxf.astype(jnp.int32)


def kernel(X, centroids):
    a = -2.0 * centroids
    xt = X.T
    csq = jnp.sum(centroids * centroids, axis=-1)[:, None]
    xsq = jnp.sum(X * X, axis=-1)[None, :]
    out = pl.pallas_call(
        _nc_kernel,
        grid=(N // BN,),
        in_specs=[
            pl.BlockSpec((K, DP), lambda i: (0, 0)),
            pl.BlockSpec((DP, BN), lambda i: (0, i)),
            pl.BlockSpec((K, 1), lambda i: (0, 0)),
            pl.BlockSpec((1, BN), lambda i: (0, i)),
        ],
        out_specs=pl.BlockSpec((BN,), lambda i: (i,)),
        out_shape=jax.ShapeDtypeStruct((N,), jnp.int32),
    )(a, xt, csq, xsq)
    return out
